# baseline (speedup 1.0000x reference)
import functools
import math

import jax
import jax.numpy as jnp
from jax import lax
from jax.experimental import pallas as pl
from jax.experimental.pallas import tpu as pltpu
from jax.experimental.pallas import tpu_sc as plsc

N_LEVELS = 16
LOG2_T = 19
T = 2 ** LOG2_T
MASK = T - 1
F = 2
B = 262144
OUTW = N_LEVELS * F
P1 = 2654435761 - 2 ** 32
GW = 8
LSTRIDE = T * F // GW

_BW = math.exp((math.log(512.0) - math.log(16.0)) / (N_LEVELS - 1))
RES = [int(16 * _BW ** i) for i in range(N_LEVELS)]

ND = 9


def _ceil16(n):
    return (n + 15) & ~15


_DL = [(RES[l] + 1) * (RES[l] + 1) for l in range(ND)]
_PL = [_ceil16(d) for d in _DL]
_DOFF = []
_off = 0
for _p in _PL:
    _DOFF.append(_off)
    _off += 2 * _p
DENSE_WORDS = _off

NW = 32
PPW = B // NW
C = 256
NCHUNK = PPW // C
NV = C // 16
BBLK = B // 128
SB = 8 * C
BCH = 1024


def _sc_body(pts, tab, out, pv, ov, dv,
             iva, sva, gva, wxa, wya,
             ivb, svb, gvb, wxb, wyb, sema, semb):
    wid = lax.axis_index("s") * 2 + lax.axis_index("c")
    lanes = lax.iota(jnp.int32, 16)
    zero16 = lanes * 0
    one16 = zero16 + 1
    bufs = ((iva, sva, gva, wxa, wya, sema),
            (ivb, svb, gvb, wxb, wyb, semb))

    for lev in range(ND):
        r1 = RES[lev] + 1
        pl_sz = _PL[lev]
        doff = _DOFF[lev]
        gbase = lev * LSTRIDE
        nbatch = (pl_sz + BCH - 1) // BCH

        for jb in range(nbatch):
            lo = jb * BCH
            cnt = min(BCH, pl_sz - lo)
            nv16 = cnt // 16

            def bg(j, c, lo=lo, r1=r1, gbase=gbase):
                lin = lanes + (lo + j * 16)
                y = lin // r1
                x = lin - y * r1
                h = (x ^ (y * P1)) & MASK
                g0 = gbase + (h >> 3) + ((h >> 7) << 4)
                iva[pl.ds(j * 16, 16)] = g0
                iva[pl.ds(BCH + j * 16, 16)] = g0 + 16
                sva[pl.ds(j * 16, 16)] = h & 7
                return c

            lax.fori_loop(0, nv16, bg, 0)
            cp0 = pltpu.async_copy(tab.at[iva.at[pl.ds(0, cnt)]],
                                   gva.at[pl.ds(0, cnt)], sema)
            cp1 = pltpu.async_copy(tab.at[iva.at[pl.ds(BCH, cnt)]],
                                   gva.at[pl.ds(BCH, cnt)], sema)
            cp0.wait()
            cp1.wait()

            def bx(j, c, lo=lo, doff=doff, pl_sz=pl_sz):
                row = lanes + j * 16
                col = sva[pl.ds(j * 16, 16)]
                f0 = plsc.load_gather(gva, [row, col])
                f1 = plsc.load_gather(gva, [row + BCH, col])
                dv[pl.ds(doff + lo + j * 16, 16)] = f0
                dv[pl.ds(doff + pl_sz + lo + j * 16, 16)] = f1
                return c

            lax.fori_loop(0, nv16, bx, 0)

    def stream_ig(lev, buf):
        iv, sv, gv, wxv, wyv, sem = buf
        rf = float(RES[lev])
        gbase = lev * LSTRIDE

        def ig(i, c):
            s = i * 16
            off = ((s >> 7) << 8) + (s & 127)
            fx = pv[pl.ds(off, 16)] * rf
            fy = pv[pl.ds(off + 128, 16)] * rf
            ix = fx.astype(jnp.int32)
            iy = fy.astype(jnp.int32)
            wxv[pl.ds(s, 16)] = fx - ix.astype(jnp.float32)
            wyv[pl.ds(s, 16)] = fy - iy.astype(jnp.float32)
            hy0 = iy * P1
            hy1 = (iy + 1) * P1
            ix1 = ix + 1
            for corner, h in enumerate((
                    (ix ^ hy0) & MASK,
                    (ix ^ hy1) & MASK,
                    (ix1 ^ hy0) & MASK,
                    (ix1 ^ hy1) & MASK,
            )):
                g0 = gbase + (h >> 3) + ((h >> 7) << 4)
                iv[pl.ds(corner * C + s, 16)] = g0
                iv[pl.ds((4 + corner) * C + s, 16)] = g0 + 16
                sv[pl.ds(corner * C + s, 16)] = h & 7
            return c

        lax.fori_loop(0, NV, ig, 0)
        cp0 = pltpu.async_copy(tab.at[iv.at[pl.ds(0, 4 * C)]],
                               gv.at[pl.ds(0, 4 * C)], sem)
        cp1 = pltpu.async_copy(tab.at[iv.at[pl.ds(4 * C, 4 * C)]],
                               gv.at[pl.ds(4 * C, 4 * C)], sem)
        return cp0, cp1

    def scatter_out(s, c0, a0, a1):
        ov[c0 // 8, s >> 7, c0 % 8, pl.ds(s & 127, 16)] = a0
        ov[c0 // 8, s >> 7, c0 % 8 + 1, pl.ds(s & 127, 16)] = a1

    def stream_cb(lev, buf):
        iv, sv, gv, wxv, wyv, sem = buf
        c0 = 2 * lev

        def cb(i, c):
            s = i * 16
            wx = wxv[pl.ds(s, 16)]
            wy = wyv[pl.ds(s, 16)]
            w00 = (1.0 - wx) * (1.0 - wy)
            w01 = (1.0 - wx) * wy
            w10 = wx * (1.0 - wy)
            w11 = wx * wy
            pt = lanes + s
            a0 = None
            a1 = None
            for corner, wc in ((0, w00), (1, w01), (2, w10), (3, w11)):
                col = sv[pl.ds(corner * C + s, 16)]
                f0 = plsc.load_gather(gv, [pt + corner * C, col])
                f1 = plsc.load_gather(gv, [pt + (4 + corner) * C, col])
                if a0 is None:
                    a0 = wc * f0
                    a1 = wc * f1
                else:
                    a0 = a0 + wc * f0
                    a1 = a1 + wc * f1
            scatter_out(s, c0, a0, a1)
            return c

        lax.fori_loop(0, NV, cb, 0)

    def dense_level(lev):
        rf = float(RES[lev])
        r1 = RES[lev] + 1
        doff = _DOFF[lev]
        pl_sz = _PL[lev]
        c0 = 2 * lev

        def dc(i, c):
            s = i * 16
            off = ((s >> 7) << 8) + (s & 127)
            fx = pv[pl.ds(off, 16)] * rf
            fy = pv[pl.ds(off + 128, 16)] * rf
            ix = fx.astype(jnp.int32)
            iy = fy.astype(jnp.int32)
            wx = fx - ix.astype(jnp.float32)
            wy = fy - iy.astype(jnp.float32)
            w00 = (1.0 - wx) * (1.0 - wy)
            w01 = (1.0 - wx) * wy
            w10 = wx * (1.0 - wy)
            w11 = wx * wy
            i00 = iy * r1 + ix + doff
            i01 = i00 + r1
            i10 = i00 + 1
            i11 = i01 + 1
            a0 = w00 * plsc.load_gather(dv, [i00])
            a1 = w00 * plsc.load_gather(dv, [i00 + pl_sz])
            a0 = a0 + w01 * plsc.load_gather(dv, [i01])
            a1 = a1 + w01 * plsc.load_gather(dv, [i01 + pl_sz])
            a0 = a0 + w10 * plsc.load_gather(dv, [i10])
            a1 = a1 + w10 * plsc.load_gather(dv, [i10 + pl_sz])
            a0 = a0 + w11 * plsc.load_gather(dv, [i11])
            a1 = a1 + w11 * plsc.load_gather(dv, [i11 + pl_sz])
            scatter_out(s, c0, a0, a1)
            return c

        lax.fori_loop(0, NV, dc, 0)

    def chunk_body(k, carry):
        base = pl.multiple_of(wid * PPW + k * C, 8)
        pltpu.sync_copy(pts.at[pl.ds(base * 2, 2 * C)], pv)

        prev_cp = stream_ig(ND, bufs[0])
        prev_lev = ND

        for lev in range(ND):
            dense_level(lev)

        for lev in range(ND + 1, N_LEVELS):
            buf = bufs[(lev - ND) % 2]
            cps = stream_ig(lev, buf)
            prev_cp[0].wait()
            prev_cp[1].wait()
            stream_cb(prev_lev, bufs[(prev_lev - ND) % 2])
            prev_cp = cps
            prev_lev = lev

        prev_cp[0].wait()
        prev_cp[1].wait()
        stream_cb(prev_lev, bufs[(prev_lev - ND) % 2])

        bb = base // 128
        for cblk in range(4):
            pltpu.sync_copy(ov.at[cblk], out.at[cblk, pl.ds(bb, C // 128)])
        return carry

    lax.fori_loop(0, NCHUNK, chunk_body, 0)


@jax.jit
def _hash_encode_sc(pts, tab):
    mesh = plsc.VectorSubcoreMesh(core_axis_name="c", subcore_axis_name="s")
    run = functools.partial(
        pl.kernel,
        mesh=mesh,
        compiler_params=pltpu.CompilerParams(
            needs_layout_passes=False, use_tc_tiling_on_sc=False
        ),
        out_type=jax.ShapeDtypeStruct((4, BBLK, 8, 128), jnp.float32),
        scratch_types=[
            pltpu.VMEM((2 * C,), jnp.float32),
            pltpu.VMEM((4, C // 128, 8, 128), jnp.float32),
            pltpu.VMEM((DENSE_WORDS,), jnp.float32),
            pltpu.VMEM((SB,), jnp.int32),
            pltpu.VMEM((4 * C,), jnp.int32),
            pltpu.VMEM((SB, GW), jnp.float32),
            pltpu.VMEM((C,), jnp.float32),
            pltpu.VMEM((C,), jnp.float32),
            pltpu.VMEM((SB,), jnp.int32),
            pltpu.VMEM((4 * C,), jnp.int32),
            pltpu.VMEM((SB, GW), jnp.float32),
            pltpu.VMEM((C,), jnp.float32),
            pltpu.VMEM((C,), jnp.float32),
            pltpu.SemaphoreType.DMA,
            pltpu.SemaphoreType.DMA,
        ],
    )(_sc_body)
    return run(pts, tab)


def kernel(points, hash_idxs, features, resolution):
    pts = jnp.transpose(points.reshape(BBLK, 128, 2), (0, 2, 1)).reshape(2 * B)
    f4 = features.reshape(N_LEVELS, T // 128, 128, F)
    tab = jnp.transpose(f4, (0, 1, 3, 2)).reshape(N_LEVELS * T * F // GW, GW)
    o4 = _hash_encode_sc(pts, tab)
    out = jnp.transpose(o4, (0, 2, 1, 3)).reshape(OUTW, B)
    return jnp.transpose(out, (1, 0))


# speedup vs baseline: 1.0088x; 1.0088x over previous
import functools
import math

import jax
import jax.numpy as jnp
from jax import lax
from jax.experimental import pallas as pl
from jax.experimental.pallas import tpu as pltpu
from jax.experimental.pallas import tpu_sc as plsc

N_LEVELS = 16
LOG2_T = 19
T = 2 ** LOG2_T
MASK = T - 1
F = 2
B = 262144
OUTW = N_LEVELS * F
P1 = 2654435761 - 2 ** 32
GW = 8
LSTRIDE = T * F // GW

_BW = math.exp((math.log(512.0) - math.log(16.0)) / (N_LEVELS - 1))
RES = [int(16 * _BW ** i) for i in range(N_LEVELS)]

ND = 9


def _ceil16(n):
    return (n + 15) & ~15


_DL = [(RES[l] + 1) * (RES[l] + 1) for l in range(ND)]
_PL = [_ceil16(d) for d in _DL]
_DOFF = []
_off = 0
for _p in _PL:
    _DOFF.append(_off)
    _off += 2 * _p
DENSE_WORDS = _off

NW = 32
PPW = B // NW
C = 256
NCHUNK = PPW // C
NV = C // 16
BBLK = B // 128
SB = 8 * C
BCH = 1024


def _sc_body(pts, tab, out, pv, ov, dv,
             iva, sva, gva, wxa, wya,
             ivb, svb, gvb, wxb, wyb, sema, semb):
    wid = lax.axis_index("s") * 2 + lax.axis_index("c")
    lanes = lax.iota(jnp.int32, 16)
    zero16 = lanes * 0
    one16 = zero16 + 1
    bufs = ((iva, sva, gva, wxa, wya, sema),
            (ivb, svb, gvb, wxb, wyb, semb))

    for lev in range(ND):
        r1 = RES[lev] + 1
        pl_sz = _PL[lev]
        doff = _DOFF[lev]
        gbase = lev * LSTRIDE
        nbatch = (pl_sz + BCH - 1) // BCH

        for jb in range(nbatch):
            lo = jb * BCH
            cnt = min(BCH, pl_sz - lo)
            nv16 = cnt // 16

            def bg(j, c, lo=lo, r1=r1, gbase=gbase):
                lin = lanes + (lo + j * 16)
                y = lin // r1
                x = lin - y * r1
                h = (x ^ (y * P1)) & MASK
                g0 = gbase + (h >> 3) + ((h >> 7) << 4)
                iva[pl.ds(j * 16, 16)] = g0
                iva[pl.ds(BCH + j * 16, 16)] = g0 + 16
                sva[pl.ds(j * 16, 16)] = h & 7
                return c

            lax.fori_loop(0, nv16, bg, 0)
            cp0 = pltpu.async_copy(tab.at[iva.at[pl.ds(0, cnt)]],
                                   gva.at[pl.ds(0, cnt)], sema)
            cp1 = pltpu.async_copy(tab.at[iva.at[pl.ds(BCH, cnt)]],
                                   gva.at[pl.ds(BCH, cnt)], sema)
            cp0.wait()
            cp1.wait()

            def bx(j, c, lo=lo, doff=doff, pl_sz=pl_sz):
                row = lanes + j * 16
                col = sva[pl.ds(j * 16, 16)]
                f0 = plsc.load_gather(gva, [row, col])
                f1 = plsc.load_gather(gva, [row + BCH, col])
                dv[pl.ds(doff + lo + j * 16, 16)] = f0
                dv[pl.ds(doff + pl_sz + lo + j * 16, 16)] = f1
                return c

            lax.fori_loop(0, nv16, bx, 0)

    def stream_ig(lev, buf):
        iv, sv, gv, wxv, wyv, sem = buf
        rf = float(RES[lev])
        gbase = lev * LSTRIDE

        def ig(s):
            off = ((s >> 7) << 8) + (s & 127)
            fx = pv[pl.ds(off, 16)] * rf
            fy = pv[pl.ds(off + 128, 16)] * rf
            ix = fx.astype(jnp.int32)
            iy = fy.astype(jnp.int32)
            wxv[pl.ds(s, 16)] = fx - ix.astype(jnp.float32)
            wyv[pl.ds(s, 16)] = fy - iy.astype(jnp.float32)
            hy0 = iy * P1
            hy1 = (iy + 1) * P1
            ix1 = ix + 1
            for corner, h in enumerate((
                    (ix ^ hy0) & MASK,
                    (ix ^ hy1) & MASK,
                    (ix1 ^ hy0) & MASK,
                    (ix1 ^ hy1) & MASK,
            )):
                g0 = gbase + (h >> 3) + ((h >> 7) << 4)
                iv[pl.ds(corner * C + s, 16)] = g0
                iv[pl.ds((4 + corner) * C + s, 16)] = g0 + 16
                sv[pl.ds(corner * C + s, 16)] = h & 7

        plsc.parallel_loop(0, C, 16, unroll=2)(ig)
        cp0 = pltpu.async_copy(tab.at[iv.at[pl.ds(0, 4 * C)]],
                               gv.at[pl.ds(0, 4 * C)], sem)
        cp1 = pltpu.async_copy(tab.at[iv.at[pl.ds(4 * C, 4 * C)]],
                               gv.at[pl.ds(4 * C, 4 * C)], sem)
        return cp0, cp1

    def scatter_out(s, c0, a0, a1):
        ov[c0 // 8, s >> 7, c0 % 8, pl.ds(s & 127, 16)] = a0
        ov[c0 // 8, s >> 7, c0 % 8 + 1, pl.ds(s & 127, 16)] = a1

    def stream_cb(lev, buf):
        iv, sv, gv, wxv, wyv, sem = buf
        c0 = 2 * lev

        def cb(s):
            wx = wxv[pl.ds(s, 16)]
            wy = wyv[pl.ds(s, 16)]
            w00 = (1.0 - wx) * (1.0 - wy)
            w01 = (1.0 - wx) * wy
            w10 = wx * (1.0 - wy)
            w11 = wx * wy
            pt = lanes + s
            f0s = []
            f1s = []
            for corner in range(4):
                col = sv[pl.ds(corner * C + s, 16)]
                f0s.append(plsc.load_gather(gv, [pt + corner * C, col]))
                f1s.append(plsc.load_gather(gv, [pt + (4 + corner) * C, col]))
            a0 = (w00 * f0s[0] + w01 * f0s[1]) + (w10 * f0s[2] + w11 * f0s[3])
            a1 = (w00 * f1s[0] + w01 * f1s[1]) + (w10 * f1s[2] + w11 * f1s[3])
            scatter_out(s, c0, a0, a1)

        plsc.parallel_loop(0, C, 16, unroll=2)(cb)

    def dense_level(lev):
        rf = float(RES[lev])
        r1 = RES[lev] + 1
        doff = _DOFF[lev]
        pl_sz = _PL[lev]
        c0 = 2 * lev

        def dc(s):
            off = ((s >> 7) << 8) + (s & 127)
            fx = pv[pl.ds(off, 16)] * rf
            fy = pv[pl.ds(off + 128, 16)] * rf
            ix = fx.astype(jnp.int32)
            iy = fy.astype(jnp.int32)
            wx = fx - ix.astype(jnp.float32)
            wy = fy - iy.astype(jnp.float32)
            w00 = (1.0 - wx) * (1.0 - wy)
            w01 = (1.0 - wx) * wy
            w10 = wx * (1.0 - wy)
            w11 = wx * wy
            i00 = iy * r1 + ix + doff
            i01 = i00 + r1
            i10 = i00 + 1
            i11 = i01 + 1
            g00 = plsc.load_gather(dv, [i00])
            g01 = plsc.load_gather(dv, [i01])
            g10 = plsc.load_gather(dv, [i10])
            g11 = plsc.load_gather(dv, [i11])
            h00 = plsc.load_gather(dv, [i00 + pl_sz])
            h01 = plsc.load_gather(dv, [i01 + pl_sz])
            h10 = plsc.load_gather(dv, [i10 + pl_sz])
            h11 = plsc.load_gather(dv, [i11 + pl_sz])
            a0 = (w00 * g00 + w01 * g01) + (w10 * g10 + w11 * g11)
            a1 = (w00 * h00 + w01 * h01) + (w10 * h10 + w11 * h11)
            scatter_out(s, c0, a0, a1)

        plsc.parallel_loop(0, C, 16, unroll=2)(dc)

    def chunk_body(k, carry):
        base = pl.multiple_of(wid * PPW + k * C, 8)
        pltpu.sync_copy(pts.at[pl.ds(base * 2, 2 * C)], pv)

        prev_cp = stream_ig(ND, bufs[0])
        prev_lev = ND

        for lev in range(ND):
            dense_level(lev)

        for lev in range(ND + 1, N_LEVELS):
            buf = bufs[(lev - ND) % 2]
            cps = stream_ig(lev, buf)
            prev_cp[0].wait()
            prev_cp[1].wait()
            stream_cb(prev_lev, bufs[(prev_lev - ND) % 2])
            prev_cp = cps
            prev_lev = lev

        prev_cp[0].wait()
        prev_cp[1].wait()
        stream_cb(prev_lev, bufs[(prev_lev - ND) % 2])

        bb = base // 128
        for cblk in range(4):
            pltpu.sync_copy(ov.at[cblk], out.at[cblk, pl.ds(bb, C // 128)])
        return carry

    lax.fori_loop(0, NCHUNK, chunk_body, 0)


@jax.jit
def _hash_encode_sc(pts, tab):
    mesh = plsc.VectorSubcoreMesh(core_axis_name="c", subcore_axis_name="s")
    run = functools.partial(
        pl.kernel,
        mesh=mesh,
        compiler_params=pltpu.CompilerParams(
            needs_layout_passes=False, use_tc_tiling_on_sc=False
        ),
        out_type=jax.ShapeDtypeStruct((4, BBLK, 8, 128), jnp.float32),
        scratch_types=[
            pltpu.VMEM((2 * C,), jnp.float32),
            pltpu.VMEM((4, C // 128, 8, 128), jnp.float32),
            pltpu.VMEM((DENSE_WORDS,), jnp.float32),
            pltpu.VMEM((SB,), jnp.int32),
            pltpu.VMEM((4 * C,), jnp.int32),
            pltpu.VMEM((SB, GW), jnp.float32),
            pltpu.VMEM((C,), jnp.float32),
            pltpu.VMEM((C,), jnp.float32),
            pltpu.VMEM((SB,), jnp.int32),
            pltpu.VMEM((4 * C,), jnp.int32),
            pltpu.VMEM((SB, GW), jnp.float32),
            pltpu.VMEM((C,), jnp.float32),
            pltpu.VMEM((C,), jnp.float32),
            pltpu.SemaphoreType.DMA,
            pltpu.SemaphoreType.DMA,
        ],
    )(_sc_body)
    return run(pts, tab)


def kernel(points, hash_idxs, features, resolution):
    pts = jnp.transpose(points.reshape(BBLK, 128, 2), (0, 2, 1)).reshape(2 * B)
    f4 = features.reshape(N_LEVELS, T // 128, 128, F)
    tab = jnp.transpose(f4, (0, 1, 3, 2)).reshape(N_LEVELS * T * F // GW, GW)
    o4 = _hash_encode_sc(pts, tab)
    out = jnp.transpose(o4, (0, 2, 1, 3)).reshape(OUTW, B)
    return jnp.transpose(out, (1, 0))


# speedup vs baseline: 1.2788x; 1.2677x over previous
import functools
import math

import jax
import jax.numpy as jnp
from jax import lax
from jax.experimental import pallas as pl
from jax.experimental.pallas import tpu as pltpu
from jax.experimental.pallas import tpu_sc as plsc

N_LEVELS = 16
LOG2_T = 19
T = 2 ** LOG2_T
MASK = T - 1
F = 2
B = 262144
OUTW = N_LEVELS * F
P1 = 2654435761 - 2 ** 32
GW = 8
LSTRIDE = T * F // GW
G4 = 4

_BW = math.exp((math.log(512.0) - math.log(16.0)) / (N_LEVELS - 1))
RES = [int(16 * _BW ** i) for i in range(N_LEVELS)]

ND = 9


def _ceil16(n):
    return (n + 15) & ~15


_DL = [(RES[l] + 1) * (RES[l] + 1) for l in range(ND)]
_PL = [_ceil16(d) for d in _DL]
_DOFF = []
_off = 0
for _p in _PL:
    _DOFF.append(_off)
    _off += 2 * _p
DENSE_WORDS = _off

NW = 32
PPW = B // NW
C = 256
NCHUNK = PPW // C
NV = C // 16
BBLK = B // 128
SB = 8 * C
BCH = 1024


def _sc_body(pts, tab, out, pv, ov, dv,
             iva, sva, gva, wxa, wya,
             ivb, svb, gvb, wxb, wyb, sema, semb):
    wid = lax.axis_index("s") * 2 + lax.axis_index("c")
    lanes = lax.iota(jnp.int32, 16)
    zero16 = lanes * 0
    one16 = zero16 + 1
    bufs = ((iva, sva, gva, wxa, wya, sema),
            (ivb, svb, gvb, wxb, wyb, semb))

    for lev in range(ND):
        r1 = RES[lev] + 1
        pl_sz = _PL[lev]
        doff = _DOFF[lev]
        gbase = lev * LSTRIDE
        nbatch = (pl_sz + BCH - 1) // BCH

        for jb in range(nbatch):
            lo = jb * BCH
            cnt = min(BCH, pl_sz - lo)
            nv16 = cnt // 16

            def bg(j, c, lo=lo, r1=r1, gbase=gbase):
                lin = lanes + (lo + j * 16)
                y = lin // r1
                x = lin - y * r1
                h = (x ^ (y * P1)) & MASK
                iva[pl.ds(j * 16, 16)] = gbase + (h >> 2)
                sva[pl.ds(j * 16, 16)] = (h & 3) * 2
                return c

            lax.fori_loop(0, nv16, bg, 0)
            pltpu.async_copy(tab.at[iva.at[pl.ds(0, cnt)]],
                             gva.at[pl.ds(0, cnt)], sema).wait()

            def bx(j, c, lo=lo, doff=doff, pl_sz=pl_sz):
                row = lanes + j * 16
                col = sva[pl.ds(j * 16, 16)]
                f0 = plsc.load_gather(gva, [row, col])
                f1 = plsc.load_gather(gva, [row, col + one16])
                dv[pl.ds(doff + lo + j * 16, 16)] = f0
                dv[pl.ds(doff + pl_sz + lo + j * 16, 16)] = f1
                return c

            lax.fori_loop(0, nv16, bx, 0)

    def stream_ig(lev, buf):
        iv, sv, gv, wxv, wyv, sem = buf
        rf = float(RES[lev])
        gbase = lev * LSTRIDE

        def ig(s):
            off = ((s >> 7) << 8) + (s & 127)
            fx = pv[pl.ds(off, 16)] * rf
            fy = pv[pl.ds(off + 128, 16)] * rf
            ix = fx.astype(jnp.int32)
            iy = fy.astype(jnp.int32)
            wxv[pl.ds(s, 16)] = fx - ix.astype(jnp.float32)
            wyv[pl.ds(s, 16)] = fy - iy.astype(jnp.float32)
            hy0 = iy * P1
            hy1 = (iy + 1) * P1
            ix1 = ix + 1
            for corner, h in enumerate((
                    (ix ^ hy0) & MASK,
                    (ix ^ hy1) & MASK,
                    (ix1 ^ hy0) & MASK,
                    (ix1 ^ hy1) & MASK,
            )):
                iv[pl.ds(corner * C + s, 16)] = gbase + (h >> 2)
                sv[pl.ds(corner * C + s, 16)] = (h & 3) * 2

        plsc.parallel_loop(0, C, 16, unroll=2)(ig)
        cp0 = pltpu.async_copy(tab.at[iv.at[pl.ds(0, 4 * C)]],
                               gv.at[pl.ds(0, 4 * C)], sem)
        return (cp0,)

    def scatter_out(s, c0, a0, a1):
        ov[c0 // 8, s >> 7, c0 % 8, pl.ds(s & 127, 16)] = a0
        ov[c0 // 8, s >> 7, c0 % 8 + 1, pl.ds(s & 127, 16)] = a1

    def stream_cb(lev, buf):
        iv, sv, gv, wxv, wyv, sem = buf
        c0 = 2 * lev

        def cb(s):
            wx = wxv[pl.ds(s, 16)]
            wy = wyv[pl.ds(s, 16)]
            w00 = (1.0 - wx) * (1.0 - wy)
            w01 = (1.0 - wx) * wy
            w10 = wx * (1.0 - wy)
            w11 = wx * wy
            pt = lanes + s
            f0s = []
            f1s = []
            for corner in range(4):
                col = sv[pl.ds(corner * C + s, 16)]
                f0s.append(plsc.load_gather(gv, [pt + corner * C, col]))
                f1s.append(plsc.load_gather(gv, [pt + corner * C, col + one16]))
            a0 = (w00 * f0s[0] + w01 * f0s[1]) + (w10 * f0s[2] + w11 * f0s[3])
            a1 = (w00 * f1s[0] + w01 * f1s[1]) + (w10 * f1s[2] + w11 * f1s[3])
            scatter_out(s, c0, a0, a1)

        plsc.parallel_loop(0, C, 16, unroll=2)(cb)

    def dense_level(lev):
        rf = float(RES[lev])
        r1 = RES[lev] + 1
        doff = _DOFF[lev]
        pl_sz = _PL[lev]
        c0 = 2 * lev

        def dc(s):
            off = ((s >> 7) << 8) + (s & 127)
            fx = pv[pl.ds(off, 16)] * rf
            fy = pv[pl.ds(off + 128, 16)] * rf
            ix = fx.astype(jnp.int32)
            iy = fy.astype(jnp.int32)
            wx = fx - ix.astype(jnp.float32)
            wy = fy - iy.astype(jnp.float32)
            w00 = (1.0 - wx) * (1.0 - wy)
            w01 = (1.0 - wx) * wy
            w10 = wx * (1.0 - wy)
            w11 = wx * wy
            i00 = iy * r1 + ix + doff
            i01 = i00 + r1
            i10 = i00 + 1
            i11 = i01 + 1
            g00 = plsc.load_gather(dv, [i00])
            g01 = plsc.load_gather(dv, [i01])
            g10 = plsc.load_gather(dv, [i10])
            g11 = plsc.load_gather(dv, [i11])
            h00 = plsc.load_gather(dv, [i00 + pl_sz])
            h01 = plsc.load_gather(dv, [i01 + pl_sz])
            h10 = plsc.load_gather(dv, [i10 + pl_sz])
            h11 = plsc.load_gather(dv, [i11 + pl_sz])
            a0 = (w00 * g00 + w01 * g01) + (w10 * g10 + w11 * g11)
            a1 = (w00 * h00 + w01 * h01) + (w10 * h10 + w11 * h11)
            scatter_out(s, c0, a0, a1)

        plsc.parallel_loop(0, C, 16, unroll=2)(dc)

    def chunk_body(k, carry):
        base = pl.multiple_of(wid * PPW + k * C, 8)
        pltpu.sync_copy(pts.at[pl.ds(base * 2, 2 * C)], pv)

        prev_cp = stream_ig(ND, bufs[0])
        prev_lev = ND

        for lev in range(ND):
            dense_level(lev)

        for lev in range(ND + 1, N_LEVELS):
            buf = bufs[(lev - ND) % 2]
            cps = stream_ig(lev, buf)
            prev_cp[0].wait()
            stream_cb(prev_lev, bufs[(prev_lev - ND) % 2])
            prev_cp = cps
            prev_lev = lev

        prev_cp[0].wait()
        stream_cb(prev_lev, bufs[(prev_lev - ND) % 2])

        bb = base // 128
        for cblk in range(4):
            pltpu.sync_copy(ov.at[cblk], out.at[cblk, pl.ds(bb, C // 128)])
        return carry

    lax.fori_loop(0, NCHUNK, chunk_body, 0)


RLW = 4096
RSPAN = N_LEVELS * T * F // NW


def _relayout_body(tabn, out, ibuf, obuf):
    wid = lax.axis_index("s") * 2 + lax.axis_index("c")
    lanes = lax.iota(jnp.int32, 16)
    base = pl.multiple_of(wid * RSPAN, 8)

    def batch(bt, carry):
        off = base + bt * RLW
        pltpu.sync_copy(tabn.at[pl.ds(off, RLW)], ibuf)

        def step(j):
            b = j >> 3
            k = j & 7
            f0 = ibuf[pl.ds(b * 256 + k * 16, 16)]
            f1 = ibuf[pl.ds(b * 256 + 128 + k * 16, 16)]
            oidx = (b * 256 + k * 32) + lanes * 2
            plsc.store_scatter(obuf, [oidx], f0)
            plsc.store_scatter(obuf, [oidx + 1], f1)

        plsc.parallel_loop(0, RLW // 32, 1, unroll=2)(step)
        pltpu.sync_copy(obuf, out.at[pl.ds(off, RLW)])
        return carry

    lax.fori_loop(0, RSPAN // RLW, batch, 0)


@jax.jit
def _relayout_sc(tabn):
    mesh = plsc.VectorSubcoreMesh(core_axis_name="c", subcore_axis_name="s")
    run = functools.partial(
        pl.kernel,
        mesh=mesh,
        compiler_params=pltpu.CompilerParams(
            needs_layout_passes=False, use_tc_tiling_on_sc=False
        ),
        out_type=jax.ShapeDtypeStruct((N_LEVELS * T * F,), jnp.float32),
        scratch_types=[
            pltpu.VMEM((RLW,), jnp.float32),
            pltpu.VMEM((RLW,), jnp.float32),
        ],
    )(_relayout_body)
    return run(tabn)


@jax.jit
def _hash_encode_sc(pts, tab):
    mesh = plsc.VectorSubcoreMesh(core_axis_name="c", subcore_axis_name="s")
    run = functools.partial(
        pl.kernel,
        mesh=mesh,
        compiler_params=pltpu.CompilerParams(
            needs_layout_passes=False, use_tc_tiling_on_sc=False
        ),
        out_type=jax.ShapeDtypeStruct((4, BBLK, 8, 128), jnp.float32),
        scratch_types=[
            pltpu.VMEM((2 * C,), jnp.float32),
            pltpu.VMEM((4, C // 128, 8, 128), jnp.float32),
            pltpu.VMEM((DENSE_WORDS,), jnp.float32),
            pltpu.VMEM((4 * C,), jnp.int32),
            pltpu.VMEM((4 * C,), jnp.int32),
            pltpu.VMEM((4 * C, GW), jnp.float32),
            pltpu.VMEM((C,), jnp.float32),
            pltpu.VMEM((C,), jnp.float32),
            pltpu.VMEM((4 * C,), jnp.int32),
            pltpu.VMEM((4 * C,), jnp.int32),
            pltpu.VMEM((4 * C, GW), jnp.float32),
            pltpu.VMEM((C,), jnp.float32),
            pltpu.VMEM((C,), jnp.float32),
            pltpu.SemaphoreType.DMA,
            pltpu.SemaphoreType.DMA,
        ],
    )(_sc_body)
    return run(pts, tab)


def kernel(points, hash_idxs, features, resolution):
    pts = jnp.transpose(points.reshape(BBLK, 128, 2), (0, 2, 1)).reshape(2 * B)
    f4 = features.reshape(N_LEVELS, T // 128, 128, F)
    tabn = jnp.transpose(f4, (0, 1, 3, 2)).reshape(N_LEVELS * T * F)
    tab = _relayout_sc(tabn).reshape(N_LEVELS * T * F // GW, GW)
    o4 = _hash_encode_sc(pts, tab)
    out = jnp.transpose(o4, (0, 2, 1, 3)).reshape(OUTW, B)
    return jnp.transpose(out, (1, 0))
